# Initial kernel scaffold; baseline (speedup 1.0000x reference)
import jax
import jax.numpy as jnp
from jax.experimental import pallas as pl


def kernel(x, edge_index, idx, W1, a1_src, a1_dst, gamma1, beta1, W2, a2_src, a2_dst, gamma2, beta2, Wd, bd):
    raise NotImplementedError("write your pallas kernel here")


# speedup vs baseline: 24.6439x; 24.6439x over previous
import functools

import jax
import jax.numpy as jnp
from jax import lax
from jax.experimental import pallas as pl
from jax.experimental.pallas import tpu as pltpu
from jax.experimental.pallas import tpu_sc as plsc

NC = 2
NS = 16
NW = NC * NS


def _mm_score_body(x_ref, w_ref, asrc_ref, adst_ref, h_ref, s_ref, d_ref):
    h = jnp.dot(x_ref[...], w_ref[...], preferred_element_type=jnp.float32)
    h_ref[...] = h
    s_ref[...] = jnp.sum(h * asrc_ref[...], axis=1)[:, None]
    d_ref[...] = jnp.sum(h * adst_ref[...], axis=1)[:, None]


def _mm_score(x, W, a_src, a_dst, rb=1000):
    n, dd = x.shape
    hh = W.shape[1]
    return pl.pallas_call(
        _mm_score_body,
        grid=(n // rb,),
        in_specs=[
            pl.BlockSpec((rb, dd), lambda i: (i, 0)),
            pl.BlockSpec((dd, hh), lambda i: (0, 0)),
            pl.BlockSpec((1, hh), lambda i: (0, 0)),
            pl.BlockSpec((1, hh), lambda i: (0, 0)),
        ],
        out_specs=[
            pl.BlockSpec((rb, hh), lambda i: (i, 0)),
            pl.BlockSpec((rb, 1), lambda i: (i, 0)),
            pl.BlockSpec((rb, 1), lambda i: (i, 0)),
        ],
        out_shape=[
            jax.ShapeDtypeStruct((n, hh), jnp.float32),
            jax.ShapeDtypeStruct((n, 1), jnp.float32),
            jax.ShapeDtypeStruct((n, 1), jnp.float32),
        ],
    )(x, W, a_src.reshape(1, -1), a_dst.reshape(1, -1))


def _agg_body(acc_ref, den0_ref, den1_ref, out_ref, st_ref):
    i = pl.program_id(0)
    a = acc_ref[0] + acc_ref[1]
    dn = den0_ref[...] + den1_ref[...]
    o = a / (dn + 1e-9)
    out_ref[...] = o

    @pl.when(i == 0)
    def _():
        st_ref[...] = jnp.zeros_like(st_ref)

    st_ref[0:1, :] += jnp.sum(o, axis=0)[None, :]
    st_ref[1:2, :] += jnp.sum(o * o, axis=0)[None, :]


def _agg(acc, den, rb=1000):
    _, n, hh = acc.shape
    nb = n // rb
    return pl.pallas_call(
        _agg_body,
        grid=(nb,),
        in_specs=[
            pl.BlockSpec((2, rb, hh), lambda i: (0, i, 0)),
            pl.BlockSpec((rb, 1), lambda i: (i, 0)),
            pl.BlockSpec((rb, 1), lambda i, nb=nb: (nb + i, 0)),
        ],
        out_specs=[
            pl.BlockSpec((rb, hh), lambda i: (i, 0)),
            pl.BlockSpec((8, hh), lambda i: (0, 0)),
        ],
        out_shape=[
            jax.ShapeDtypeStruct((n, hh), jnp.float32),
            jax.ShapeDtypeStruct((8, hh), jnp.float32),
        ],
    )(acc, den, den)


def _bn(o_ref, st_ref, g_ref, b_ref, n):
    mu = st_ref[0:1, :] / n
    var = st_ref[1:2, :] / n - mu * mu
    xb = g_ref[...] * (o_ref[...] - mu) * lax.rsqrt(var + 1e-5) + b_ref[...]
    return jnp.maximum(xb, 0.0)


def _bn_mm_score_body(o_ref, st_ref, g_ref, b_ref, w_ref, asrc_ref, adst_ref,
                      h_ref, s_ref, d_ref, *, n):
    xb = _bn(o_ref, st_ref, g_ref, b_ref, n)
    h = jnp.dot(xb, w_ref[...], preferred_element_type=jnp.float32)
    h_ref[...] = h
    s_ref[...] = jnp.sum(h * asrc_ref[...], axis=1)[:, None]
    d_ref[...] = jnp.sum(h * adst_ref[...], axis=1)[:, None]


def _bn_mm_score(o, st, gamma, beta, W, a_src, a_dst, rb=1000):
    n, hh = o.shape
    return pl.pallas_call(
        functools.partial(_bn_mm_score_body, n=float(n)),
        grid=(n // rb,),
        in_specs=[
            pl.BlockSpec((rb, hh), lambda i: (i, 0)),
            pl.BlockSpec((8, hh), lambda i: (0, 0)),
            pl.BlockSpec((1, hh), lambda i: (0, 0)),
            pl.BlockSpec((1, hh), lambda i: (0, 0)),
            pl.BlockSpec((hh, hh), lambda i: (0, 0)),
            pl.BlockSpec((1, hh), lambda i: (0, 0)),
            pl.BlockSpec((1, hh), lambda i: (0, 0)),
        ],
        out_specs=[
            pl.BlockSpec((rb, hh), lambda i: (i, 0)),
            pl.BlockSpec((rb, 1), lambda i: (i, 0)),
            pl.BlockSpec((rb, 1), lambda i: (i, 0)),
        ],
        out_shape=[
            jax.ShapeDtypeStruct((n, hh), jnp.float32),
            jax.ShapeDtypeStruct((n, 1), jnp.float32),
            jax.ShapeDtypeStruct((n, 1), jnp.float32),
        ],
    )(o, st, gamma.reshape(1, -1), beta.reshape(1, -1), W,
      a_src.reshape(1, -1), a_dst.reshape(1, -1))


def _bn_dense_body(o_ref, st_ref, g_ref, b_ref, wd_ref, bd_ref, out_ref, *, n):
    xb = _bn(o_ref, st_ref, g_ref, b_ref, n)
    out_ref[...] = jnp.dot(xb, wd_ref[...],
                           preferred_element_type=jnp.float32) + bd_ref[...]


def _bn_dense(o, st, gamma, beta, Wd, bd, nstat, rb=1024):
    n, hh = o.shape
    cdim = Wd.shape[1]
    return pl.pallas_call(
        functools.partial(_bn_dense_body, n=float(nstat)),
        grid=(n // rb,),
        in_specs=[
            pl.BlockSpec((rb, hh), lambda i: (i, 0)),
            pl.BlockSpec((8, hh), lambda i: (0, 0)),
            pl.BlockSpec((1, hh), lambda i: (0, 0)),
            pl.BlockSpec((1, hh), lambda i: (0, 0)),
            pl.BlockSpec((hh, cdim), lambda i: (0, 0)),
            pl.BlockSpec((1, cdim), lambda i: (0, 0)),
        ],
        out_specs=pl.BlockSpec((rb, cdim), lambda i: (i, 0)),
        out_shape=jax.ShapeDtypeStruct((n, cdim), jnp.float32),
    )(o, st, gamma.reshape(1, -1), beta.reshape(1, -1), Wd, bd.reshape(1, -1))


def _edge_pass(src2, dst2, s, d, h):
    nchunks = src2.shape[0]
    n = s.shape[0]
    hh = h.shape[1]
    nv = hh // 16
    rpt = n // NS
    np_den = ((n + 16 * NS - 1) // (16 * NS)) * 16 * NS
    dpt = np_den // NS
    mesh = plsc.VectorSubcoreMesh(core_axis_name="c", subcore_axis_name="s")

    @functools.partial(
        pl.kernel,
        out_type=[
            jax.ShapeDtypeStruct((NC * n, hh), jnp.float32),
            jax.ShapeDtypeStruct((NC * n,), jnp.float32),
        ],
        mesh=mesh,
        scratch_types=[
            pltpu.VMEM((n,), jnp.float32),
            pltpu.VMEM((n,), jnp.float32),
            pltpu.VMEM((128, hh), jnp.float32),
            pltpu.VMEM((128,), jnp.int32),
            pltpu.VMEM((128,), jnp.int32),
            pltpu.VMEM((128,), jnp.float32),
            pltpu.VMEM((dpt,), jnp.float32),
            pltpu.VMEM_SHARED((n, hh), jnp.float32),
            pltpu.VMEM_SHARED((np_den,), jnp.float32),
            pltpu.SemaphoreType.DMA,
        ],
        compiler_params=pltpu.CompilerParams(needs_layout_passes=False),
    )
    def k(src_hbm, dst_hbm, s_hbm, d_hbm, h_hbm, acc_out, den_out,
          s_v, d_v, rows, csrc, cdst, exbuf, zden, acc_sh, den_sh, sem):
        cid = lax.axis_index("c")
        sid = lax.axis_index("s")
        wid = sid * NC + cid

        def zrow(r, _):
            for v in range(nv):
                rows[r, pl.ds(v * 16, 16)] = jnp.zeros((16,), jnp.float32)
            return 0
        lax.fori_loop(0, 128, zrow, 0)

        def zd(j, _):
            zden[pl.ds(j * 16, 16)] = jnp.zeros((16,), jnp.float32)
            return 0
        lax.fori_loop(0, dpt // 16, zd, 0)

        cq, cr = divmod(n // 128, NS)
        cbase = cq * sid + jnp.minimum(sid, cr)
        ccount = cq + jnp.where(sid < cr, 1, 0)
        tail = n - (n // 128) * 128

        def zacc(kk, _):
            pltpu.sync_copy(rows, acc_sh.at[pl.ds((cbase + kk) * 128, 128)])
            return 0
        lax.fori_loop(0, ccount, zacc, 0)
        if tail:
            @pl.when(sid == NS - 1)
            def _():
                pltpu.sync_copy(rows.at[pl.ds(0, tail)],
                                acc_sh.at[pl.ds(n - tail, tail)])
        pltpu.sync_copy(zden, den_sh.at[pl.ds(sid * dpt, dpt)])

        pltpu.sync_copy(s_hbm, s_v)
        pltpu.sync_copy(d_hbm, d_v)
        plsc.subcore_barrier()

        q, r = divmod(nchunks, NW)
        base = q * wid + jnp.minimum(wid, r)
        count = q + jnp.where(wid < r, 1, 0)

        def chunk(kk, _):
            c = base + kk
            pltpu.sync_copy(src_hbm.at[c], csrc)
            pltpu.sync_copy(dst_hbm.at[c], cdst)
            g = pltpu.async_copy(h_hbm.at[csrc], rows, sem)
            for j in range(8):
                si = csrc[pl.ds(j * 16, 16)]
                di = cdst[pl.ds(j * 16, 16)]
                z = plsc.load_gather(s_v, [si]) + plsc.load_gather(d_v, [di])
                e = jnp.maximum(z, 0.2 * z)
                exbuf[pl.ds(j * 16, 16)] = jnp.exp(e)
            pltpu.sync_copy(exbuf, den_sh.at[cdst], add=True)
            g.wait()

            def wrow(rr, _):
                bex = plsc.load_gather(
                    exbuf, [jnp.zeros((16,), jnp.int32) + rr])
                for v in range(nv):
                    sl = pl.ds(v * 16, 16)
                    rows[rr, sl] = rows[rr, sl] * bex
                return 0
            lax.fori_loop(0, 128, wrow, 0)
            pltpu.sync_copy(rows, acc_sh.at[cdst], add=True)
            return 0

        lax.fori_loop(0, count, chunk, 0)
        plsc.subcore_barrier()

        def facc(kk, _):
            off = (cbase + kk) * 128
            pltpu.sync_copy(acc_sh.at[pl.ds(off, 128)], rows)
            pltpu.sync_copy(rows, acc_out.at[pl.ds(cid * n + off, 128)])
            return 0
        lax.fori_loop(0, ccount, facc, 0)
        if tail:
            @pl.when(sid == NS - 1)
            def _():
                pltpu.sync_copy(acc_sh.at[pl.ds(n - tail, tail)],
                                rows.at[pl.ds(0, tail)])
                pltpu.sync_copy(rows.at[pl.ds(0, tail)],
                                acc_out.at[pl.ds(cid * n + n - tail, tail)])

        dlast = n - (NS - 1) * dpt
        @pl.when(sid < NS - 1)
        def _():
            pltpu.sync_copy(den_sh.at[pl.ds(sid * dpt, dpt)], zden)
            pltpu.sync_copy(zden, den_out.at[pl.ds(cid * n + sid * dpt, dpt)])

        @pl.when(sid == NS - 1)
        def _():
            pltpu.sync_copy(den_sh.at[pl.ds((NS - 1) * dpt, dlast)],
                            zden.at[pl.ds(0, dlast)])
            pltpu.sync_copy(zden.at[pl.ds(0, dlast)],
                            den_out.at[pl.ds(cid * n + (NS - 1) * dpt, dlast)])

    return k(src2, dst2, s, d, h)


def _gather_rows(tab, idx):
    n, cdim = tab.shape
    b = idx.shape[0]
    per = b // NW
    mesh = plsc.VectorSubcoreMesh(core_axis_name="c", subcore_axis_name="s")

    @functools.partial(
        pl.kernel,
        out_type=jax.ShapeDtypeStruct((b, cdim), jnp.float32),
        mesh=mesh,
        scratch_types=[
            pltpu.VMEM((per,), jnp.int32),
            pltpu.VMEM((per, cdim), jnp.float32),
            pltpu.SemaphoreType.DMA,
        ],
    )
    def k(tab_hbm, idx_hbm, out_hbm, idx_v, rows_v, sem):
        cid = lax.axis_index("c")
        sid = lax.axis_index("s")
        wid = sid * NC + cid
        base = wid * per
        pltpu.sync_copy(idx_hbm.at[pl.ds(base, per)], idx_v)
        pltpu.async_copy(tab_hbm.at[idx_v], rows_v, sem).wait()
        pltpu.sync_copy(rows_v, out_hbm.at[pl.ds(base, per)])

    return k(tab, idx)


def kernel(x, edge_index, idx, W1, a1_src, a1_dst, gamma1, beta1,
           W2, a2_src, a2_dst, gamma2, beta2, Wd, bd):
    n = x.shape[0]
    e = edge_index.shape[1]
    hh = W1.shape[1]

    src2 = edge_index[0].astype(jnp.int32).reshape(e // 128, 128)
    dst2 = edge_index[1].astype(jnp.int32).reshape(e // 128, 128)

    h1, s1, d1 = _mm_score(x, W1, a1_src, a1_dst)
    acc1, den1 = _edge_pass(src2, dst2, s1.reshape(-1), d1.reshape(-1), h1)
    out1, st1 = _agg(acc1.reshape(NC, n, hh), den1.reshape(NC * n, 1))

    h2, s2, d2 = _bn_mm_score(out1, st1, gamma1, beta1, W2, a2_src, a2_dst)
    acc2, den2 = _edge_pass(src2, dst2, s2.reshape(-1), d2.reshape(-1), h2)
    out2, st2 = _agg(acc2.reshape(NC, n, hh), den2.reshape(NC * n, 1))

    g = _gather_rows(out2, idx.astype(jnp.int32))
    return _bn_dense(g, st2, gamma2, beta2, Wd, bd, nstat=n)


# speedup vs baseline: 29.5388x; 1.1986x over previous
import functools

import jax
import jax.numpy as jnp
from jax import lax
from jax.experimental import pallas as pl
from jax.experimental.pallas import tpu as pltpu
from jax.experimental.pallas import tpu_sc as plsc

NC = 2
NS = 16
NW = NC * NS


def _mm_score_body(x_ref, w_ref, asrc_ref, adst_ref, h_ref, s_ref, d_ref):
    h = jnp.dot(x_ref[...], w_ref[...], preferred_element_type=jnp.float32)
    h_ref[...] = h
    s_ref[...] = jnp.sum(h * asrc_ref[...], axis=1)[:, None]
    d_ref[...] = jnp.sum(h * adst_ref[...], axis=1)[:, None]


def _mm_score(x, W, a_src, a_dst, rb=1000):
    n, dd = x.shape
    hh = W.shape[1]
    return pl.pallas_call(
        _mm_score_body,
        grid=(n // rb,),
        in_specs=[
            pl.BlockSpec((rb, dd), lambda i: (i, 0)),
            pl.BlockSpec((dd, hh), lambda i: (0, 0)),
            pl.BlockSpec((1, hh), lambda i: (0, 0)),
            pl.BlockSpec((1, hh), lambda i: (0, 0)),
        ],
        out_specs=[
            pl.BlockSpec((rb, hh), lambda i: (i, 0)),
            pl.BlockSpec((rb, 1), lambda i: (i, 0)),
            pl.BlockSpec((rb, 1), lambda i: (i, 0)),
        ],
        out_shape=[
            jax.ShapeDtypeStruct((n, hh), jnp.float32),
            jax.ShapeDtypeStruct((n, 1), jnp.float32),
            jax.ShapeDtypeStruct((n, 1), jnp.float32),
        ],
    )(x, W, a_src.reshape(1, -1), a_dst.reshape(1, -1))


def _agg_body(acc_ref, den0_ref, den1_ref, out_ref, st_ref):
    i = pl.program_id(0)
    a = acc_ref[0] + acc_ref[1]
    dn = den0_ref[...] + den1_ref[...]
    o = a / (dn + 1e-9)
    out_ref[...] = o

    @pl.when(i == 0)
    def _():
        st_ref[...] = jnp.zeros_like(st_ref)

    st_ref[0:1, :] += jnp.sum(o, axis=0)[None, :]
    st_ref[1:2, :] += jnp.sum(o * o, axis=0)[None, :]


def _agg(acc, den, rb=1000):
    _, n, hh = acc.shape
    nb = n // rb
    return pl.pallas_call(
        _agg_body,
        grid=(nb,),
        in_specs=[
            pl.BlockSpec((2, rb, hh), lambda i: (0, i, 0)),
            pl.BlockSpec((rb, 1), lambda i: (i, 0)),
            pl.BlockSpec((rb, 1), lambda i, nb=nb: (nb + i, 0)),
        ],
        out_specs=[
            pl.BlockSpec((rb, hh), lambda i: (i, 0)),
            pl.BlockSpec((8, hh), lambda i: (0, 0)),
        ],
        out_shape=[
            jax.ShapeDtypeStruct((n, hh), jnp.float32),
            jax.ShapeDtypeStruct((8, hh), jnp.float32),
        ],
    )(acc, den, den)


def _bn(o_ref, st_ref, g_ref, b_ref, n):
    mu = st_ref[0:1, :] / n
    var = st_ref[1:2, :] / n - mu * mu
    xb = g_ref[...] * (o_ref[...] - mu) * lax.rsqrt(var + 1e-5) + b_ref[...]
    return jnp.maximum(xb, 0.0)


def _bn_mm_score_body(o_ref, st_ref, g_ref, b_ref, w_ref, asrc_ref, adst_ref,
                      h_ref, s_ref, d_ref, *, n):
    xb = _bn(o_ref, st_ref, g_ref, b_ref, n)
    h = jnp.dot(xb, w_ref[...], preferred_element_type=jnp.float32)
    h_ref[...] = h
    s_ref[...] = jnp.sum(h * asrc_ref[...], axis=1)[:, None]
    d_ref[...] = jnp.sum(h * adst_ref[...], axis=1)[:, None]


def _bn_mm_score(o, st, gamma, beta, W, a_src, a_dst, rb=1000):
    n, hh = o.shape
    return pl.pallas_call(
        functools.partial(_bn_mm_score_body, n=float(n)),
        grid=(n // rb,),
        in_specs=[
            pl.BlockSpec((rb, hh), lambda i: (i, 0)),
            pl.BlockSpec((8, hh), lambda i: (0, 0)),
            pl.BlockSpec((1, hh), lambda i: (0, 0)),
            pl.BlockSpec((1, hh), lambda i: (0, 0)),
            pl.BlockSpec((hh, hh), lambda i: (0, 0)),
            pl.BlockSpec((1, hh), lambda i: (0, 0)),
            pl.BlockSpec((1, hh), lambda i: (0, 0)),
        ],
        out_specs=[
            pl.BlockSpec((rb, hh), lambda i: (i, 0)),
            pl.BlockSpec((rb, 1), lambda i: (i, 0)),
            pl.BlockSpec((rb, 1), lambda i: (i, 0)),
        ],
        out_shape=[
            jax.ShapeDtypeStruct((n, hh), jnp.float32),
            jax.ShapeDtypeStruct((n, 1), jnp.float32),
            jax.ShapeDtypeStruct((n, 1), jnp.float32),
        ],
    )(o, st, gamma.reshape(1, -1), beta.reshape(1, -1), W,
      a_src.reshape(1, -1), a_dst.reshape(1, -1))


def _bn_dense_body(o_ref, st_ref, g_ref, b_ref, wd_ref, bd_ref, out_ref, *, n):
    xb = _bn(o_ref, st_ref, g_ref, b_ref, n)
    out_ref[...] = jnp.dot(xb, wd_ref[...],
                           preferred_element_type=jnp.float32) + bd_ref[...]


def _bn_dense(o, st, gamma, beta, Wd, bd, nstat, rb=1024):
    n, hh = o.shape
    cdim = Wd.shape[1]
    return pl.pallas_call(
        functools.partial(_bn_dense_body, n=float(nstat)),
        grid=(n // rb,),
        in_specs=[
            pl.BlockSpec((rb, hh), lambda i: (i, 0)),
            pl.BlockSpec((8, hh), lambda i: (0, 0)),
            pl.BlockSpec((1, hh), lambda i: (0, 0)),
            pl.BlockSpec((1, hh), lambda i: (0, 0)),
            pl.BlockSpec((hh, cdim), lambda i: (0, 0)),
            pl.BlockSpec((1, cdim), lambda i: (0, 0)),
        ],
        out_specs=pl.BlockSpec((rb, cdim), lambda i: (i, 0)),
        out_shape=jax.ShapeDtypeStruct((n, cdim), jnp.float32),
    )(o, st, gamma.reshape(1, -1), beta.reshape(1, -1), Wd, bd.reshape(1, -1))


def _edge_pass(src2, dst2, s, d, h):
    nchunks = src2.shape[0]
    n = s.shape[0]
    hh = h.shape[1]
    nv = hh // 16
    np_den = ((n + 16 * NS - 1) // (16 * NS)) * 16 * NS
    dpt = np_den // NS
    mesh = plsc.VectorSubcoreMesh(core_axis_name="c", subcore_axis_name="s")

    @functools.partial(
        pl.kernel,
        out_type=[
            jax.ShapeDtypeStruct((NC * n, hh), jnp.float32),
            jax.ShapeDtypeStruct((NC * n,), jnp.float32),
        ],
        mesh=mesh,
        scratch_types=[
            pltpu.VMEM((n,), jnp.float32),
            pltpu.VMEM((n,), jnp.float32),
            pltpu.VMEM((128, hh), jnp.float32),
            pltpu.VMEM((128,), jnp.int32),
            pltpu.VMEM((128,), jnp.int32),
            pltpu.VMEM((128,), jnp.float32),
            pltpu.VMEM((dpt,), jnp.float32),
            pltpu.VMEM_SHARED((n, hh), jnp.float32),
            pltpu.VMEM_SHARED((np_den,), jnp.float32),
            pltpu.SemaphoreType.DMA,
        ],
        compiler_params=pltpu.CompilerParams(needs_layout_passes=False),
    )
    def k(src_hbm, dst_hbm, s_hbm, d_hbm, h_hbm, acc_out, den_out,
          s_v, d_v, rows, csrc, cdst, exbuf, zden, acc_sh, den_sh, sem):
        cid = lax.axis_index("c")
        sid = lax.axis_index("s")
        wid = sid * NC + cid

        def zrow(r, _):
            for v in range(nv):
                rows[r, pl.ds(v * 16, 16)] = jnp.zeros((16,), jnp.float32)
            return 0
        lax.fori_loop(0, 128, zrow, 0)

        def zd(j, _):
            zden[pl.ds(j * 16, 16)] = jnp.zeros((16,), jnp.float32)
            return 0
        lax.fori_loop(0, dpt // 16, zd, 0)

        cq, cr = divmod(n // 128, NS)
        cbase = cq * sid + jnp.minimum(sid, cr)
        ccount = cq + jnp.where(sid < cr, 1, 0)
        tail = n - (n // 128) * 128

        def zacc(kk, _):
            pltpu.sync_copy(rows, acc_sh.at[pl.ds((cbase + kk) * 128, 128)])
            return 0
        lax.fori_loop(0, ccount, zacc, 0)
        if tail:
            @pl.when(sid == NS - 1)
            def _():
                pltpu.sync_copy(rows.at[pl.ds(0, tail)],
                                acc_sh.at[pl.ds(n - tail, tail)])
        pltpu.sync_copy(zden, den_sh.at[pl.ds(sid * dpt, dpt)])

        pltpu.sync_copy(s_hbm, s_v)
        pltpu.sync_copy(d_hbm, d_v)
        plsc.subcore_barrier()

        q, r = divmod(nchunks, NW)
        base = q * wid + jnp.minimum(wid, r)
        count = q + jnp.where(wid < r, 1, 0)

        def chunk(kk, _):
            c = base + kk
            pltpu.sync_copy(src_hbm.at[c], csrc)
            g = pltpu.async_copy(h_hbm.at[csrc], rows, sem)
            pltpu.sync_copy(dst_hbm.at[c], cdst)
            for j in range(8):
                si = csrc[pl.ds(j * 16, 16)]
                di = cdst[pl.ds(j * 16, 16)]
                z = plsc.load_gather(s_v, [si]) + plsc.load_gather(d_v, [di])
                e = jnp.maximum(z, 0.2 * z)
                exbuf[pl.ds(j * 16, 16)] = jnp.exp(e)
            pltpu.sync_copy(exbuf, den_sh.at[cdst], add=True)
            g.wait()

            def wrow(rr, _):
                r2 = rr * 2
                bex0 = plsc.load_gather(
                    exbuf, [jnp.zeros((16,), jnp.int32) + r2])
                bex1 = plsc.load_gather(
                    exbuf, [jnp.zeros((16,), jnp.int32) + (r2 + 1)])
                for v in range(nv):
                    sl = pl.ds(v * 16, 16)
                    rows[r2, sl] = rows[r2, sl] * bex0
                    rows[r2 + 1, sl] = rows[r2 + 1, sl] * bex1
                return 0
            lax.fori_loop(0, 64, wrow, 0)
            pltpu.sync_copy(rows, acc_sh.at[cdst], add=True)
            return 0

        lax.fori_loop(0, count, chunk, 0)
        plsc.subcore_barrier()

        def facc(kk, _):
            off = (cbase + kk) * 128
            pltpu.sync_copy(acc_sh.at[pl.ds(off, 128)], rows)
            pltpu.sync_copy(rows, acc_out.at[pl.ds(cid * n + off, 128)])
            return 0
        lax.fori_loop(0, ccount, facc, 0)
        if tail:
            @pl.when(sid == NS - 1)
            def _():
                pltpu.sync_copy(acc_sh.at[pl.ds(n - tail, tail)],
                                rows.at[pl.ds(0, tail)])
                pltpu.sync_copy(rows.at[pl.ds(0, tail)],
                                acc_out.at[pl.ds(cid * n + n - tail, tail)])

        dlast = n - (NS - 1) * dpt

        @pl.when(sid < NS - 1)
        def _():
            pltpu.sync_copy(den_sh.at[pl.ds(sid * dpt, dpt)], zden)
            pltpu.sync_copy(zden, den_out.at[pl.ds(cid * n + sid * dpt, dpt)])

        @pl.when(sid == NS - 1)
        def _():
            pltpu.sync_copy(den_sh.at[pl.ds((NS - 1) * dpt, dlast)],
                            zden.at[pl.ds(0, dlast)])
            pltpu.sync_copy(zden.at[pl.ds(0, dlast)],
                            den_out.at[pl.ds(cid * n + (NS - 1) * dpt, dlast)])

    return k(src2, dst2, s, d, h)


def _gather_rows(tab, idx):
    n, cdim = tab.shape
    b = idx.shape[0]
    per = b // NW
    mesh = plsc.VectorSubcoreMesh(core_axis_name="c", subcore_axis_name="s")

    @functools.partial(
        pl.kernel,
        out_type=jax.ShapeDtypeStruct((b, cdim), jnp.float32),
        mesh=mesh,
        scratch_types=[
            pltpu.VMEM((per,), jnp.int32),
            pltpu.VMEM((per, cdim), jnp.float32),
            pltpu.SemaphoreType.DMA,
        ],
    )
    def k(tab_hbm, idx_hbm, out_hbm, idx_v, rows_v, sem):
        cid = lax.axis_index("c")
        sid = lax.axis_index("s")
        wid = sid * NC + cid
        base = wid * per
        pltpu.sync_copy(idx_hbm.at[pl.ds(base, per)], idx_v)
        pltpu.async_copy(tab_hbm.at[idx_v], rows_v, sem).wait()
        pltpu.sync_copy(rows_v, out_hbm.at[pl.ds(base, per)])

    return k(tab, idx)


def kernel(x, edge_index, idx, W1, a1_src, a1_dst, gamma1, beta1,
           W2, a2_src, a2_dst, gamma2, beta2, Wd, bd):
    n = x.shape[0]
    e = edge_index.shape[1]
    hh = W1.shape[1]

    src2 = edge_index[0].astype(jnp.int32).reshape(e // 128, 128)
    dst2 = edge_index[1].astype(jnp.int32).reshape(e // 128, 128)

    h1, s1, d1 = _mm_score(x, W1, a1_src, a1_dst)
    acc1, den1 = _edge_pass(src2, dst2, s1.reshape(-1), d1.reshape(-1), h1)
    out1, st1 = _agg(acc1.reshape(NC, n, hh), den1.reshape(NC * n, 1))

    h2, s2, d2 = _bn_mm_score(out1, st1, gamma1, beta1, W2, a2_src, a2_dst)
    acc2, den2 = _edge_pass(src2, dst2, s2.reshape(-1), d2.reshape(-1), h2)
    out2, st2 = _agg(acc2.reshape(NC, n, hh), den2.reshape(NC * n, 1))

    g = _gather_rows(out2, idx.astype(jnp.int32))
    return _bn_dense(g, st2, gamma2, beta2, Wd, bd, nstat=n)
